# Initial kernel scaffold; baseline (speedup 1.0000x reference)
import jax
import jax.numpy as jnp
from jax.experimental import pallas as pl


def kernel(f_atoms, f_bonds, edge_index, graph_ids, W_i, b_i, W_h, b_h, W_o, b_o, W_f1, b_f1, W_f2, b_f2):
    raise NotImplementedError("write your pallas kernel here")


# speedup vs baseline: 3.1683x; 3.1683x over previous
import functools

import jax
import jax.numpy as jnp
from jax import lax
from jax.experimental import pallas as pl
from jax.experimental.pallas import tpu as pltpu
from jax.experimental.pallas import tpu_sc as plsc

N = 10000
E = 320000
DA = 128
DE = 16
H = 128
G = 64

NC = 2
NS = 16
NW = NC * NS
PER_W = E // NW
C = 80
N_CHUNKS = PER_W // C
NP_ = 10240
ZR = NP_ // NS

_F32 = jnp.float32


def _p_body(x_ref, w_ref, o_ref):
    o_ref[...] = jnp.dot(x_ref[...], w_ref[...], preferred_element_type=_F32)


def _tc_p(f_atoms, w):
    return pl.pallas_call(
        _p_body,
        out_shape=jax.ShapeDtypeStruct((N, H), _F32),
    )(f_atoms, w)


def _qb_body(x_ref, w_ref, b_ref, o_ref):
    o_ref[...] = (jnp.dot(x_ref[...], w_ref[...], preferred_element_type=_F32)
                  + b_ref[...])


def _tc_qb(f_bonds, w, b):
    be = 16000
    return pl.pallas_call(
        _qb_body,
        grid=(E // be,),
        in_specs=[
            pl.BlockSpec((be, DE), lambda i: (i, 0)),
            pl.BlockSpec((DE, H), lambda i: (0, 0)),
            pl.BlockSpec((1, H), lambda i: (0, 0)),
        ],
        out_specs=pl.BlockSpec((be, H), lambda i: (i, 0)),
        out_shape=jax.ShapeDtypeStruct((E, H), _F32),
    )(f_bonds, w, b)


def _a_body(acc_ref, w_ref, b_ref, o_ref):
    a = acc_ref[0] + acc_ref[1]
    o_ref[...] = jnp.dot(a, w_ref[...], preferred_element_type=_F32) + b_ref[...]


def _tc_a(acc, w, b):
    return pl.pallas_call(
        _a_body,
        out_shape=jax.ShapeDtypeStruct((NP_, H), _F32),
    )(acc, w, b)


def _head_body(fa_ref, acc_ref, gid_ref, woa_ref, woh_ref, bo_ref,
               wf1_ref, bf1_ref, wf2_ref, bf2_ref, o_ref):
    a_in = (acc_ref[0] + acc_ref[1])[:N]
    atom = jnp.maximum(
        jnp.dot(fa_ref[...], woa_ref[...], preferred_element_type=_F32)
        + jnp.dot(a_in, woh_ref[...], preferred_element_type=_F32)
        + bo_ref[...], 0.0)
    gid = gid_ref[...]
    onehot = (gid == lax.broadcasted_iota(jnp.int32, (G, N), 0)).astype(_F32)
    mol = jnp.dot(onehot, atom, preferred_element_type=_F32)
    counts = jnp.sum(onehot, axis=1, keepdims=True)
    mol = mol / jnp.maximum(counts, 1.0)
    ffn = jnp.maximum(jnp.dot(mol, wf1_ref[...], preferred_element_type=_F32)
                      + bf1_ref[...], 0.0)
    o_ref[...] = (jnp.dot(ffn, wf2_ref[...], preferred_element_type=_F32)
                  + bf2_ref[...])


def _tc_head(f_atoms, acc, gid_row, woa, woh, bo, wf1, bf1, wf2, bf2):
    return pl.pallas_call(
        _head_body,
        out_shape=jax.ShapeDtypeStruct((G, 1), _F32),
    )(f_atoms, acc, gid_row, woa, woh, bo, wf1, bf1, wf2, bf2)


def _make_sc_pass(write_rows: bool):
    mesh = plsc.VectorSubcoreMesh(core_axis_name="c", subcore_axis_name="s")
    if write_rows:
        out_type = (jax.ShapeDtypeStruct((E, H), _F32),
                    jax.ShapeDtypeStruct((NC, NP_, H), _F32))
    else:
        out_type = jax.ShapeDtypeStruct((NC, NP_, H), _F32)

    @functools.partial(
        pl.kernel,
        out_type=out_type,
        mesh=mesh,
        scratch_types=[
            pltpu.VMEM((C,), jnp.int32),
            pltpu.VMEM((C,), jnp.int32),
            pltpu.VMEM((C, H), _F32),
            pltpu.VMEM((C, H), _F32),
            pltpu.VMEM_SHARED((NP_, H), _F32),
            pltpu.SemaphoreType.DMA,
        ],
    )
    def sc_pass(table_hbm, lin_hbm, src_hbm, dst_hbm, zrows_hbm, *refs):
        if write_rows:
            (rows_out_hbm, acc_hbm, idx_s, idx_d, rows_g, rows_l,
             acc_sh, sem) = refs
        else:
            acc_hbm, idx_s, idx_d, rows_g, rows_l, acc_sh, sem = refs
        cid = lax.axis_index("c")
        sid = lax.axis_index("s")
        wid = cid * NS + sid

        pltpu.sync_copy(zrows_hbm, acc_sh.at[pl.ds(sid * ZR, ZR)])
        plsc.subcore_barrier()

        @pl.loop(0, N_CHUNKS)
        def _chunk(ci):
            base = wid * PER_W + ci * C
            pltpu.sync_copy(src_hbm.at[pl.ds(base, C)], idx_s)
            pltpu.sync_copy(dst_hbm.at[pl.ds(base, C)], idx_d)
            gcp = pltpu.async_copy(table_hbm.at[idx_s], rows_g, sem)
            pltpu.sync_copy(lin_hbm.at[pl.ds(base, C)], rows_l)
            gcp.wait()

            @pl.loop(0, C)
            def _row(r):
                for j in range(H // 16):
                    s = pl.ds(j * 16, 16)
                    rows_l[r, s] = jnp.maximum(rows_g[r, s] + rows_l[r, s],
                                               0.0)

            if write_rows:
                pltpu.sync_copy(rows_l, rows_out_hbm.at[pl.ds(base, C)])
            pltpu.sync_copy(rows_l, acc_sh.at[idx_d], add=True)

        plsc.subcore_barrier()
        r0 = sid * ZR
        pltpu.sync_copy(acc_sh.at[pl.ds(r0, ZR)],
                        acc_hbm.at[cid, pl.ds(r0, ZR)])

    return sc_pass


_sc_pass0 = _make_sc_pass(write_rows=True)
_sc_pass1 = _make_sc_pass(write_rows=False)


def kernel(f_atoms, f_bonds, edge_index, graph_ids,
           W_i, b_i, W_h, b_h, W_o, b_o, W_f1, b_f1, W_f2, b_f2):
    src = edge_index[0]
    dst = edge_index[1]
    zrows = jnp.zeros((ZR, H), _F32)
    gid_row = graph_ids.reshape(1, N)

    P = _tc_p(f_atoms, W_i[:DA])
    Qb = _tc_qb(f_bonds, W_i[DA:], b_i.reshape(1, H))
    h0, acc = _sc_pass0(P, Qb, src, dst, zrows)
    for _ in range(2):
        A = _tc_a(acc, W_h, b_h.reshape(1, H))
        acc = _sc_pass1(A, h0, src, dst, zrows)
    return _tc_head(f_atoms, acc, gid_row, W_o[:DA], W_o[DA:],
                    b_o.reshape(1, H), W_f1, b_f1.reshape(1, H),
                    W_f2, b_f2.reshape(1, 1))


# speedup vs baseline: 4.1885x; 1.3220x over previous
import functools

import jax
import jax.numpy as jnp
from jax import lax
from jax.experimental import pallas as pl
from jax.experimental.pallas import tpu as pltpu
from jax.experimental.pallas import tpu_sc as plsc

N = 10000
E = 320000
DA = 128
DE = 16
H = 128
G = 64

NC = 2
NS = 16
NW = NC * NS
PER_W = E // NW
C = 40
N_CHUNKS = PER_W // C
NP_ = 10240
ZR = NP_ // NS

_F32 = jnp.float32


def _p_body(x_ref, w_ref, o_ref):
    o_ref[...] = jnp.dot(x_ref[...], w_ref[...], preferred_element_type=_F32)


def _tc_p(f_atoms, w):
    return pl.pallas_call(
        _p_body,
        out_shape=jax.ShapeDtypeStruct((N, H), _F32),
    )(f_atoms, w)


def _qb_body(x_ref, w_ref, b_ref, o_ref):
    o_ref[...] = (jnp.dot(x_ref[...], w_ref[...], preferred_element_type=_F32)
                  + b_ref[...])


def _tc_qb(f_bonds, w, b):
    be = 16000
    return pl.pallas_call(
        _qb_body,
        grid=(E // be,),
        in_specs=[
            pl.BlockSpec((be, DE), lambda i: (i, 0)),
            pl.BlockSpec((DE, H), lambda i: (0, 0)),
            pl.BlockSpec((1, H), lambda i: (0, 0)),
        ],
        out_specs=pl.BlockSpec((be, H), lambda i: (i, 0)),
        out_shape=jax.ShapeDtypeStruct((E, H), _F32),
    )(f_bonds, w, b)


def _a_body(acc_ref, w_ref, b_ref, o_ref):
    a = acc_ref[0] + acc_ref[1]
    o_ref[...] = jnp.dot(a, w_ref[...], preferred_element_type=_F32) + b_ref[...]


def _tc_a(acc, w, b):
    return pl.pallas_call(
        _a_body,
        out_shape=jax.ShapeDtypeStruct((NP_, H), _F32),
    )(acc, w, b)


def _head_body(fa_ref, acc_ref, gid_ref, woa_ref, woh_ref, bo_ref,
               wf1_ref, bf1_ref, wf2_ref, bf2_ref, o_ref):
    a_in = (acc_ref[0] + acc_ref[1])[:N]
    atom = jnp.maximum(
        jnp.dot(fa_ref[...], woa_ref[...], preferred_element_type=_F32)
        + jnp.dot(a_in, woh_ref[...], preferred_element_type=_F32)
        + bo_ref[...], 0.0)
    gid = gid_ref[...]
    onehot = (gid == lax.broadcasted_iota(jnp.int32, (G, N), 0)).astype(_F32)
    mol = jnp.dot(onehot, atom, preferred_element_type=_F32)
    counts = jnp.sum(onehot, axis=1, keepdims=True)
    mol = mol / jnp.maximum(counts, 1.0)
    ffn = jnp.maximum(jnp.dot(mol, wf1_ref[...], preferred_element_type=_F32)
                      + bf1_ref[...], 0.0)
    o_ref[...] = (jnp.dot(ffn, wf2_ref[...], preferred_element_type=_F32)
                  + bf2_ref[...])


def _tc_head(f_atoms, acc, gid_row, woa, woh, bo, wf1, bf1, wf2, bf2):
    return pl.pallas_call(
        _head_body,
        out_shape=jax.ShapeDtypeStruct((G, 1), _F32),
    )(f_atoms, acc, gid_row, woa, woh, bo, wf1, bf1, wf2, bf2)


def _make_sc_pass(write_rows: bool):
    mesh = plsc.VectorSubcoreMesh(core_axis_name="c", subcore_axis_name="s")
    nout = 2 if write_rows else 1
    if write_rows:
        out_type = (jax.ShapeDtypeStruct((E, H), _F32),
                    jax.ShapeDtypeStruct((NC, NP_, H), _F32))
    else:
        out_type = jax.ShapeDtypeStruct((NC, NP_, H), _F32)

    @functools.partial(
        pl.kernel,
        out_type=out_type,
        mesh=mesh,
        scratch_types=[
            pltpu.VMEM((C,), jnp.int32),
            pltpu.VMEM((C,), jnp.int32),
            pltpu.VMEM((C,), jnp.int32),
            pltpu.VMEM((C,), jnp.int32),
            pltpu.VMEM((C, H), _F32),
            pltpu.VMEM((C, H), _F32),
            pltpu.VMEM((C, H), _F32),
            pltpu.VMEM((C, H), _F32),
            pltpu.VMEM_SHARED((NP_, H), _F32),
            pltpu.SemaphoreType.DMA,
            pltpu.SemaphoreType.DMA,
            pltpu.SemaphoreType.DMA,
            pltpu.SemaphoreType.DMA,
            pltpu.SemaphoreType.DMA,
            pltpu.SemaphoreType.DMA,
            pltpu.SemaphoreType.DMA,
            pltpu.SemaphoreType.DMA,
        ],
    )
    def sc_pass(table_hbm, lin_hbm, src_hbm, dst_hbm, zrows_hbm, *refs):
        if write_rows:
            rows_out_hbm, acc_hbm = refs[0], refs[1]
            rest = refs[2:]
        else:
            acc_hbm = refs[0]
            rest = refs[1:]
        (is0, is1, id0, id1, g0, g1, l0, l1, acc_sh,
         si0, si1, sg0, sg1, sl0, sl1, so0, so1) = rest
        cid = lax.axis_index("c")
        sid = lax.axis_index("s")
        wid = cid * NS + sid
        ebase = wid * PER_W

        pltpu.sync_copy(zrows_hbm, acc_sh.at[pl.ds(sid * ZR, ZR)])

        IS = (is0, is1)
        ID = (id0, id1)
        Gs = (g0, g1)
        Ls = (l0, l1)
        SI = (si0, si1)
        SG = (sg0, sg1)
        SL = (sl0, sl1)
        SO = (so0, so1)

        def issue_idx(cc, b):
            pltpu.async_copy(src_hbm.at[pl.ds(ebase + cc * C, C)], IS[b],
                             SI[b])
            pltpu.async_copy(dst_hbm.at[pl.ds(ebase + cc * C, C)], ID[b],
                             SI[b])

        def wait_idx(cc, b):
            pltpu.make_async_copy(src_hbm.at[pl.ds(ebase + cc * C, C)],
                                  IS[b], SI[b]).wait()
            pltpu.make_async_copy(dst_hbm.at[pl.ds(ebase + cc * C, C)],
                                  ID[b], SI[b]).wait()

        def issue_data(cc, b):
            pltpu.async_copy(table_hbm.at[IS[b]], Gs[b], SG[b])
            pltpu.async_copy(lin_hbm.at[pl.ds(ebase + cc * C, C)], Ls[b],
                             SL[b])

        def wait_in(cc, b):
            pltpu.make_async_copy(table_hbm.at[IS[b]], Gs[b], SG[b]).wait()
            pltpu.make_async_copy(lin_hbm.at[pl.ds(ebase + cc * C, C)],
                                  Ls[b], SL[b]).wait()

        def compute(b):
            g = Gs[b]
            l = Ls[b]

            @pl.loop(0, C)
            def _row(r):
                for j in range(H // 16):
                    s = pl.ds(j * 16, 16)
                    l[r, s] = jnp.maximum(g[r, s] + l[r, s], 0.0)

        def out(cc, b):
            if write_rows:
                pltpu.async_copy(Ls[b], rows_out_hbm.at[pl.ds(ebase + cc * C,
                                                              C)], SO[b])
            pltpu.sync_copy(Ls[b], acc_sh.at[ID[b]], add=True)

        def drain_out(b):
            if write_rows:
                pltpu.make_async_copy(lin_hbm.at[pl.ds(0, C)], Gs[b],
                                      SO[b]).wait()

        plsc.subcore_barrier()

        npairs = N_CHUNKS // 2
        issue_idx(0, 0)
        issue_idx(1, 1)
        wait_idx(0, 0)
        issue_data(0, 0)

        @pl.loop(0, npairs)
        def _pair(i):
            c0 = 2 * i

            @pl.when(i > 0)
            def _():
                drain_out(1)
                issue_idx(c0 + 1, 1)

            wait_idx(c0 + 1, 1)
            issue_data(c0 + 1, 1)
            wait_in(c0, 0)
            compute(0)
            out(c0, 0)
            wait_in(c0 + 1, 1)
            compute(1)
            drain_out(0)

            @pl.when(i < npairs - 1)
            def _():
                issue_idx(c0 + 2, 0)

            out(c0 + 1, 1)

            @pl.when(i < npairs - 1)
            def _():
                wait_idx(c0 + 2, 0)
                issue_data(c0 + 2, 0)

        drain_out(1)

        plsc.subcore_barrier()
        r0 = sid * ZR
        pltpu.sync_copy(acc_sh.at[pl.ds(r0, ZR)],
                        acc_hbm.at[cid, pl.ds(r0, ZR)])

    return sc_pass


_sc_pass0 = _make_sc_pass(write_rows=True)
_sc_pass1 = _make_sc_pass(write_rows=False)


def kernel(f_atoms, f_bonds, edge_index, graph_ids,
           W_i, b_i, W_h, b_h, W_o, b_o, W_f1, b_f1, W_f2, b_f2):
    src = edge_index[0]
    dst = edge_index[1]
    zrows = jnp.zeros((ZR, H), _F32)
    gid_row = graph_ids.reshape(1, N)

    P = _tc_p(f_atoms, W_i[:DA])
    Qb = _tc_qb(f_bonds, W_i[DA:], b_i.reshape(1, H))
    h0, acc = _sc_pass0(P, Qb, src, dst, zrows)
    for _ in range(2):
        A = _tc_a(acc, W_h, b_h.reshape(1, H))
        acc = _sc_pass1(A, h0, src, dst, zrows)
    return _tc_head(f_atoms, acc, gid_row, W_o[:DA], W_o[DA:],
                    b_o.reshape(1, H), W_f1, b_f1.reshape(1, H),
                    W_f2, b_f2.reshape(1, 1))


# speedup vs baseline: 4.7256x; 1.1282x over previous
import functools

import jax
import jax.numpy as jnp
from jax import lax
from jax.experimental import pallas as pl
from jax.experimental.pallas import tpu as pltpu
from jax.experimental.pallas import tpu_sc as plsc

N = 10000
E = 320000
DA = 128
DE = 16
H = 128
G = 64

NC = 2
NS = 16
NW = NC * NS
PER_W = E // NW
C = 80
N_CHUNKS = PER_W // C
NP_ = 10240
ZR = NP_ // NS

_F32 = jnp.float32


def _p_body(x_ref, w_ref, o_ref):
    o_ref[...] = jnp.dot(x_ref[...], w_ref[...], preferred_element_type=_F32)


def _tc_p(f_atoms, w):
    return pl.pallas_call(
        _p_body,
        out_shape=jax.ShapeDtypeStruct((N, H), _F32),
    )(f_atoms, w)


def _qb_body(x_ref, w_ref, b_ref, o_ref):
    o_ref[...] = (jnp.dot(x_ref[...], w_ref[...], preferred_element_type=_F32)
                  + b_ref[...])


def _tc_qb(f_bonds, w, b):
    be = 16000
    return pl.pallas_call(
        _qb_body,
        grid=(E // be,),
        in_specs=[
            pl.BlockSpec((be, DE), lambda i: (i, 0)),
            pl.BlockSpec((DE, H), lambda i: (0, 0)),
            pl.BlockSpec((1, H), lambda i: (0, 0)),
        ],
        out_specs=pl.BlockSpec((be, H), lambda i: (i, 0)),
        out_shape=jax.ShapeDtypeStruct((E, H), _F32),
    )(f_bonds, w, b)


def _a_body(acc_ref, w_ref, b_ref, o_ref):
    a = acc_ref[0] + acc_ref[1]
    o_ref[...] = jnp.dot(a, w_ref[...], preferred_element_type=_F32) + b_ref[...]


def _tc_a(acc, w, b):
    return pl.pallas_call(
        _a_body,
        out_shape=jax.ShapeDtypeStruct((NP_, H), _F32),
    )(acc, w, b)


def _head_body(fa_ref, acc_ref, gid_ref, woa_ref, woh_ref, bo_ref,
               wf1_ref, bf1_ref, wf2_ref, bf2_ref, o_ref):
    a_in = (acc_ref[0] + acc_ref[1])[:N]
    atom = jnp.maximum(
        jnp.dot(fa_ref[...], woa_ref[...], preferred_element_type=_F32)
        + jnp.dot(a_in, woh_ref[...], preferred_element_type=_F32)
        + bo_ref[...], 0.0)
    gid = gid_ref[...]
    onehot = (gid == lax.broadcasted_iota(jnp.int32, (G, N), 0)).astype(_F32)
    mol = jnp.dot(onehot, atom, preferred_element_type=_F32)
    counts = jnp.sum(onehot, axis=1, keepdims=True)
    mol = mol / jnp.maximum(counts, 1.0)
    ffn = jnp.maximum(jnp.dot(mol, wf1_ref[...], preferred_element_type=_F32)
                      + bf1_ref[...], 0.0)
    o_ref[...] = (jnp.dot(ffn, wf2_ref[...], preferred_element_type=_F32)
                  + bf2_ref[...])


def _tc_head(f_atoms, acc, gid_row, woa, woh, bo, wf1, bf1, wf2, bf2):
    return pl.pallas_call(
        _head_body,
        out_shape=jax.ShapeDtypeStruct((G, 1), _F32),
    )(f_atoms, acc, gid_row, woa, woh, bo, wf1, bf1, wf2, bf2)


def _make_sc_pass(write_rows: bool):
    mesh = plsc.VectorSubcoreMesh(core_axis_name="c", subcore_axis_name="s")
    nout = 2 if write_rows else 1
    if write_rows:
        out_type = (jax.ShapeDtypeStruct((E, H), _F32),
                    jax.ShapeDtypeStruct((NC, NP_, H), _F32))
    else:
        out_type = jax.ShapeDtypeStruct((NC, NP_, H), _F32)

    @functools.partial(
        pl.kernel,
        out_type=out_type,
        mesh=mesh,
        scratch_types=[
            pltpu.VMEM((C,), jnp.int32),
            pltpu.VMEM((C,), jnp.int32),
            pltpu.VMEM((C,), jnp.int32),
            pltpu.VMEM((C,), jnp.int32),
            pltpu.VMEM((C, H), _F32),
            pltpu.VMEM((C, H), _F32),
            pltpu.VMEM((C, H), _F32),
            pltpu.VMEM((C, H), _F32),
            pltpu.VMEM_SHARED((NP_, H), _F32),
            pltpu.SemaphoreType.DMA,
            pltpu.SemaphoreType.DMA,
            pltpu.SemaphoreType.DMA,
            pltpu.SemaphoreType.DMA,
            pltpu.SemaphoreType.DMA,
            pltpu.SemaphoreType.DMA,
            pltpu.SemaphoreType.DMA,
            pltpu.SemaphoreType.DMA,
        ],
    )
    def sc_pass(table_hbm, lin_hbm, src_hbm, dst_hbm, *refs):
        if write_rows:
            rows_out_hbm, acc_hbm = refs[0], refs[1]
            rest = refs[2:]
        else:
            acc_hbm = refs[0]
            rest = refs[1:]
        (is0, is1, id0, id1, g0, g1, l0, l1, acc_sh,
         si0, si1, sg0, sg1, sl0, sl1, so0, so1) = rest
        cid = lax.axis_index("c")
        sid = lax.axis_index("s")
        wid = cid * NS + sid
        ebase = wid * PER_W

        IS = (is0, is1)
        ID = (id0, id1)
        Gs = (g0, g1)
        Ls = (l0, l1)
        SI = (si0, si1)
        SG = (sg0, sg1)
        SL = (sl0, sl1)
        SO = (so0, so1)

        def issue_idx(cc, b):
            pltpu.async_copy(src_hbm.at[pl.ds(ebase + cc * C, C)], IS[b],
                             SI[b])
            pltpu.async_copy(dst_hbm.at[pl.ds(ebase + cc * C, C)], ID[b],
                             SI[b])

        def wait_idx(cc, b):
            pltpu.make_async_copy(src_hbm.at[pl.ds(ebase + cc * C, C)],
                                  IS[b], SI[b]).wait()
            pltpu.make_async_copy(dst_hbm.at[pl.ds(ebase + cc * C, C)],
                                  ID[b], SI[b]).wait()

        def issue_data(cc, b):
            pltpu.async_copy(table_hbm.at[IS[b]], Gs[b], SG[b])
            pltpu.async_copy(lin_hbm.at[pl.ds(ebase + cc * C, C)], Ls[b],
                             SL[b])

        def wait_in(cc, b):
            pltpu.make_async_copy(table_hbm.at[IS[b]], Gs[b], SG[b]).wait()
            pltpu.make_async_copy(lin_hbm.at[pl.ds(ebase + cc * C, C)],
                                  Ls[b], SL[b]).wait()

        def compute(b):
            g = Gs[b]
            l = Ls[b]

            @pl.loop(0, C)
            def _row(r):
                for j in range(H // 16):
                    s = pl.ds(j * 16, 16)
                    l[r, s] = jnp.maximum(g[r, s] + l[r, s], 0.0)

        def out(cc, b):
            if write_rows:
                pltpu.async_copy(Ls[b], rows_out_hbm.at[pl.ds(ebase + cc * C,
                                                              C)], SO[b])
            pltpu.sync_copy(Ls[b], acc_sh.at[ID[b]], add=True)

        def drain_out(b):
            if write_rows:
                pltpu.make_async_copy(lin_hbm.at[pl.ds(0, C)], Gs[b],
                                      SO[b]).wait()

        @pl.loop(0, C)
        def _zrow(r):
            for j in range(H // 16):
                l0[r, pl.ds(j * 16, 16)] = jnp.zeros((16,), _F32)

        for k in range(ZR // C):
            pltpu.sync_copy(l0, acc_sh.at[pl.ds(sid * ZR + k * C, C)])

        plsc.subcore_barrier()

        npairs = N_CHUNKS // 2
        issue_idx(0, 0)
        issue_idx(1, 1)
        wait_idx(0, 0)
        issue_data(0, 0)

        @pl.loop(0, npairs)
        def _pair(i):
            c0 = 2 * i

            @pl.when(i > 0)
            def _():
                drain_out(1)
                issue_idx(c0 + 1, 1)

            wait_idx(c0 + 1, 1)
            issue_data(c0 + 1, 1)
            wait_in(c0, 0)
            compute(0)
            out(c0, 0)
            wait_in(c0 + 1, 1)
            compute(1)
            drain_out(0)
            issue_idx(c0 + 2, 0)
            out(c0 + 1, 1)
            wait_idx(c0 + 2, 0)
            issue_data(c0 + 2, 0)

        drain_out(1)
        wait_in(N_CHUNKS - 1, 0)
        compute(0)
        out(N_CHUNKS - 1, 0)
        drain_out(0)

        plsc.subcore_barrier()
        r0 = sid * ZR
        pltpu.sync_copy(acc_sh.at[pl.ds(r0, ZR)],
                        acc_hbm.at[cid, pl.ds(r0, ZR)])

    return sc_pass


_sc_pass0 = _make_sc_pass(write_rows=True)
_sc_pass1 = _make_sc_pass(write_rows=False)


def kernel(f_atoms, f_bonds, edge_index, graph_ids,
           W_i, b_i, W_h, b_h, W_o, b_o, W_f1, b_f1, W_f2, b_f2):
    src = edge_index[0]
    dst = edge_index[1]
    gid_row = graph_ids.reshape(1, N)

    P = _tc_p(f_atoms, W_i[:DA])
    Qb = _tc_qb(f_bonds, W_i[DA:], b_i.reshape(1, H))
    h0, acc = _sc_pass0(P, Qb, src, dst)
    for _ in range(2):
        A = _tc_a(acc, W_h, b_h.reshape(1, H))
        acc = _sc_pass1(A, h0, src, dst)
    return _tc_head(f_atoms, acc, gid_row, W_o[:DA], W_o[DA:],
                    b_o.reshape(1, H), W_f1, b_f1.reshape(1, H),
                    W_f2, b_f2.reshape(1, 1))
